# Initial kernel scaffold; baseline (speedup 1.0000x reference)
import jax
import jax.numpy as jnp
from jax.experimental import pallas as pl


def kernel(z, time_features, pos, edge_index, params):
    raise NotImplementedError("write your pallas kernel here")


# speedup vs baseline: 3.0756x; 3.0756x over previous
import functools

import jax
import jax.numpy as jnp
from jax import lax
from jax.experimental import pallas as pl
from jax.experimental.pallas import tpu as pltpu
from jax.experimental.pallas import tpu_sc as plsc

N = 10000
NP = 10240
E = 320000
D = 128
PW = 8
PS = 16

NC = 2
NS = 16
NW = NC * NS
EPW = E // NW
CHUNK = 80
NCHUNK = EPW // CHUNK

RPT = NP // NS
ZROWS = 128

EB = 2000
NB = 1024

@functools.cache
def _mesh():
  return plsc.VectorSubcoreMesh(
      core_axis_name="c", subcore_axis_name="s", num_cores=NC, num_subcores=NS)


def _ln(x, g, b, eps=1e-5):
  mu = jnp.mean(x, axis=-1, keepdims=True)
  xc = x - mu
  var = jnp.mean(xc * xc, axis=-1, keepdims=True)
  return xc * lax.rsqrt(var + eps) * g + b


def _embed_body(x_ref, we_ref, be_ref, wa_ref, wb_ref, h_ref, a_ref, b_ref):
  h = jnp.dot(x_ref[...], we_ref[...], preferred_element_type=jnp.float32)
  h = h + be_ref[...]
  h_ref[...] = h
  a_ref[...] = jnp.dot(h, wa_ref[...], preferred_element_type=jnp.float32)
  b_ref[...] = jnp.dot(h, wb_ref[...], preferred_element_type=jnp.float32)


def _embed(x8, we8, be, wa, wb):
  grid = NP // NB
  full = lambda i: (0, 0)
  return pl.pallas_call(
      _embed_body,
      grid=(grid,),
      in_specs=[
          pl.BlockSpec((NB, 8), lambda i: (i, 0)),
          pl.BlockSpec((8, D), full),
          pl.BlockSpec((1, D), full),
          pl.BlockSpec((D, D), full),
          pl.BlockSpec((D, D), full),
      ],
      out_specs=[
          pl.BlockSpec((NB, D), lambda i: (i, 0)),
          pl.BlockSpec((NB, D), lambda i: (i, 0)),
          pl.BlockSpec((NB, D), lambda i: (i, 0)),
      ],
      out_shape=[
          jax.ShapeDtypeStruct((NP, D), jnp.float32),
          jax.ShapeDtypeStruct((NP, D), jnp.float32),
          jax.ShapeDtypeStruct((NP, D), jnp.float32),
      ],
  )(x8, we8, be, wa, wb)


def _gather_body(a_hbm, b_hbm, p_hbm, dst_hbm, src_hbm, pre_hbm, pd_hbm,
                 idx_d, idx_s, buf_a, buf_b, buf_pd, pos_v):
  wid = lax.axis_index("s") * NC + lax.axis_index("c")
  base = wid * EPW

  pltpu.sync_copy(p_hbm, pos_v)

  def zrow(r, _):
    buf_pd[pl.ds(r * 16, 16)] = jnp.zeros((16,), jnp.float32)
    return ()

  lax.fori_loop(0, CHUNK * PW // 16, zrow, (), unroll=False)

  def chunk(c, _):
    eo = base + c * CHUNK
    pltpu.sync_copy(dst_hbm.at[pl.ds(eo, CHUNK)], idx_d)
    pltpu.sync_copy(src_hbm.at[pl.ds(eo, CHUNK)], idx_s)
    pltpu.sync_copy(a_hbm.at[idx_d], buf_a)
    pltpu.sync_copy(b_hbm.at[idx_s], buf_b)

    def row(r, _):
      for j in range(D // 16):
        sl = pl.ds(j * 16, 16)
        buf_a[r, sl] = buf_a[r, sl] + buf_b[r, sl]
      return ()

    lax.fori_loop(0, CHUNK, row, (), unroll=False)
    pltpu.sync_copy(buf_a, pre_hbm.at[pl.ds(eo, CHUNK)])

    iota = lax.iota(jnp.int32, 16)

    def grp(g, _):
      d16 = idx_d[pl.ds(g * 16, 16)]
      s16 = idx_s[pl.ds(g * 16, 16)]
      oaddr = (g * 16 + iota) * PW
      for k in range(3):
        val = (plsc.load_gather(pos_v, [d16 + k * NP])
               - plsc.load_gather(pos_v, [s16 + k * NP]))
        plsc.store_scatter(buf_pd, [oaddr + k], val)
      return ()

    lax.fori_loop(0, CHUNK // 16, grp, (), unroll=False)
    pltpu.sync_copy(buf_pd, pd_hbm.at[pl.ds(eo * PW, CHUNK * PW)])
    return ()

  lax.fori_loop(0, NCHUNK, chunk, (), unroll=False)


@functools.cache
def _gather_kernel():
  return pl.kernel(
      _gather_body,
      out_type=[
          jax.ShapeDtypeStruct((E, D), jnp.float32),
          jax.ShapeDtypeStruct((E * PW,), jnp.float32),
      ],
      mesh=_mesh(),
      scratch_types=[
          pltpu.VMEM((CHUNK,), jnp.int32),
          pltpu.VMEM((CHUNK,), jnp.int32),
          pltpu.VMEM((CHUNK, D), jnp.float32),
          pltpu.VMEM((CHUNK, D), jnp.float32),
          pltpu.VMEM((CHUNK * PW,), jnp.float32),
          pltpu.VMEM((3 * NP,), jnp.float32),
      ],
      compiler_params=pltpu.CompilerParams(needs_layout_passes=False),
  )


def _gather(A, B, ppl, dst, src):
  if False:
    P = ppl.reshape(3, NP).T
    pd = jnp.zeros((E, PW), jnp.float32).at[:, 0:3].set(P[dst] - P[src])
    return A[dst] + B[src], pd
  pre, pd_flat = _gather_kernel()(A, B, ppl, dst, src)
  return pre, pd_flat.reshape(E, PW)


def _edge_body(pre_ref, pd_ref, w2_ref, b2_ref, g1_ref, bl1_ref, g2_ref,
               bl2_ref, wp1_ref, bp1_ref, gp_ref, blp_ref, w1c_ref, b1_ref,
               wp2_ref, bp2_ref, one3_ref, m_ref, po_ref):
  pd = pd_ref[...]
  d2 = jnp.sum(pd * pd, axis=-1, keepdims=True) + 1e-12
  dists = jnp.sqrt(d2)
  m0 = pre_ref[...] + dists * w1c_ref[...] + b1_ref[...]
  m1 = jnp.maximum(_ln(m0, g1_ref[...], bl1_ref[...]), 0.0)
  m2 = jnp.dot(m1, w2_ref[...], preferred_element_type=jnp.float32) + b2_ref[...]
  m2 = jnp.maximum(_ln(m2, g2_ref[...], bl2_ref[...]), 0.0)
  ph = jnp.dot(m2, wp1_ref[...], preferred_element_type=jnp.float32) + bp1_ref[...]
  ph = jnp.maximum(_ln(ph, gp_ref[...], blp_ref[...]), 0.0)
  pw = jnp.sum(ph * wp2_ref[...] + bp2_ref[...], axis=-1, keepdims=True)
  m_ref[...] = m2
  pdw = pd * pw
  po_ref[...] = jnp.concatenate([pdw, jnp.zeros_like(pdw)], axis=-1) + one3_ref[...]


def _edge_mlp(pre, pd, w2, b2, g1, bl1, g2, bl2, wp1, bp1, gp, blp, w1c, b1,
              wp2, bp2b, one3):
  grid = E // EB
  full = lambda i: (0, 0)
  row = lambda i: (i, 0)
  return pl.pallas_call(
      _edge_body,
      grid=(grid,),
      in_specs=[
          pl.BlockSpec((EB, D), row),
          pl.BlockSpec((EB, PW), row),
          pl.BlockSpec((D, D), full),
          pl.BlockSpec((1, D), full),
          pl.BlockSpec((1, D), full),
          pl.BlockSpec((1, D), full),
          pl.BlockSpec((1, D), full),
          pl.BlockSpec((1, D), full),
          pl.BlockSpec((D, D), full),
          pl.BlockSpec((1, D), full),
          pl.BlockSpec((1, D), full),
          pl.BlockSpec((1, D), full),
          pl.BlockSpec((1, D), full),
          pl.BlockSpec((1, D), full),
          pl.BlockSpec((1, D), full),
          pl.BlockSpec((1, D), full),
          pl.BlockSpec((1, PS), full),
      ],
      out_specs=[
          pl.BlockSpec((EB, D), row),
          pl.BlockSpec((EB, PS), row),
      ],
      out_shape=[
          jax.ShapeDtypeStruct((E, D), jnp.float32),
          jax.ShapeDtypeStruct((E, PS), jnp.float32),
      ],
  )(pre, pd, w2, b2, g1, bl1, g2, bl2, wp1, bp1, gp, blp, w1c, b1, wp2,
    bp2b, one3)


def _scatter_m_body(m_hbm, dst_hbm, z_hbm, mp_hbm, idx, buf_m, acc_m):
  cid = lax.axis_index("c")
  sid = lax.axis_index("s")
  wid = sid * NC + cid

  pltpu.sync_copy(z_hbm, acc_m.at[pl.ds(sid * RPT, RPT)])
  plsc.subcore_barrier()

  base = wid * EPW

  def chunk(c, _):
    eo = base + c * CHUNK
    pltpu.sync_copy(dst_hbm.at[pl.ds(eo, CHUNK)], idx)
    pltpu.sync_copy(m_hbm.at[pl.ds(eo, CHUNK)], buf_m)
    pltpu.sync_copy(buf_m, acc_m.at[idx], add=True)
    return ()

  lax.fori_loop(0, NCHUNK, chunk, (), unroll=False)
  plsc.subcore_barrier()

  ro = sid * RPT
  pltpu.sync_copy(acc_m.at[pl.ds(ro, RPT)], mp_hbm.at[cid, pl.ds(ro, RPT)])


@functools.cache
def _scatter_m_kernel():
  return pl.kernel(
      _scatter_m_body,
      out_type=jax.ShapeDtypeStruct((NC, NP, D), jnp.float32),
      mesh=_mesh(),
      scratch_types=[
          pltpu.VMEM((CHUNK,), jnp.int32),
          pltpu.VMEM((CHUNK, D), jnp.float32),
          pltpu.VMEM_SHARED((NP, D), jnp.float32),
      ],
      compiler_params=pltpu.CompilerParams(needs_layout_passes=False),
  )


def _scatter_p_body(po_hbm, dst_hbm, z_hbm, pp_hbm, idx, buf_p, buf128, acc_p):
  cid = lax.axis_index("c")
  sid = lax.axis_index("s")
  wid = sid * NC + cid

  pltpu.sync_copy(z_hbm, acc_p.at[pl.ds(sid * RPT, RPT)])
  pltpu.sync_copy(z_hbm.at[pl.ds(0, CHUNK)], buf128)
  plsc.subcore_barrier()

  base = wid * EPW

  def chunk(c, _):
    eo = base + c * CHUNK
    pltpu.sync_copy(dst_hbm.at[pl.ds(eo, CHUNK)], idx)
    pltpu.sync_copy(po_hbm.at[pl.ds(eo, CHUNK)], buf_p)

    def row(r, _):
      buf128[r, pl.ds(0, PS)] = buf_p[r, pl.ds(0, PS)]
      return ()

    lax.fori_loop(0, CHUNK, row, (), unroll=False)
    pltpu.sync_copy(buf128, acc_p.at[idx], add=True)
    return ()

  lax.fori_loop(0, NCHUNK, chunk, (), unroll=False)
  plsc.subcore_barrier()

  ro = sid * RPT
  pltpu.sync_copy(acc_p.at[pl.ds(ro, RPT)], pp_hbm.at[cid, pl.ds(ro, RPT)])


@functools.cache
def _scatter_p_kernel():
  return pl.kernel(
      _scatter_p_body,
      out_type=jax.ShapeDtypeStruct((NC, NP, D), jnp.float32),
      mesh=_mesh(),
      scratch_types=[
          pltpu.VMEM((CHUNK,), jnp.int32),
          pltpu.VMEM((CHUNK, PS), jnp.float32),
          pltpu.VMEM((CHUNK, D), jnp.float32),
          pltpu.VMEM_SHARED((NP, D), jnp.float32),
      ],
      compiler_params=pltpu.CompilerParams(needs_layout_passes=False),
  )


def _scatter(m, po, dst):
  if False:
    mp = jax.ops.segment_sum(m, dst, num_segments=NP)
    pp = jax.ops.segment_sum(po, dst, num_segments=NP)
    z = jnp.zeros_like
    return jnp.stack([mp, z(mp)]), jnp.stack([pp, z(pp)])
  zm = jnp.zeros((RPT, D), jnp.float32)
  mp = _scatter_m_kernel()(m, dst, zm)
  pp = _scatter_p_kernel()(po, dst, zm)
  return mp, pp


def _upd_body(first, last, h_ref, p16_ref, mp0_ref, mp1_ref, pp0_ref, pp1_ref,
              cnt_ref, wu1a_ref, wu1b_ref, bu1_ref, gu1_ref, blu1_ref,
              wu2_ref, bu2_ref, gu2_ref, blu2_ref, mask3_ref, wna_ref,
              wnb_ref, bn_ref, *out_refs):
  h = h_ref[...]
  msg = mp0_ref[...] + mp1_ref[...]
  pa = pp0_ref[...] + pp1_ref[...]
  if first:
    cnt = jnp.maximum(pa[:, 3:4], 1.0)
  else:
    cnt = cnt_ref[...][:, 0:1]
  u = (jnp.dot(h, wu1a_ref[...], preferred_element_type=jnp.float32)
       + jnp.dot(msg, wu1b_ref[...], preferred_element_type=jnp.float32)
       + bu1_ref[...])
  u = jnp.maximum(_ln(u, gu1_ref[...], blu1_ref[...]), 0.0)
  u = jnp.dot(u, wu2_ref[...], preferred_element_type=jnp.float32) + bu2_ref[...]
  u = jnp.maximum(_ln(u, gu2_ref[...], blu2_ref[...]), 0.0)
  hn = h + u
  pn = p16_ref[...] + (pa[:, 0:PW] * mask3_ref[...]) / cnt
  if last:
    o = jnp.maximum(
        jnp.dot(hn, wna_ref[...], preferred_element_type=jnp.float32)
        + bn_ref[...], 0.0)
    o8 = jnp.dot(o, wnb_ref[...][:, 0:8], preferred_element_type=jnp.float32)
    out_refs[0][...] = o8
  else:
    out_refs[0][...] = hn
    out_refs[1][...] = pn
    out_refs[2][...] = jnp.dot(hn, wna_ref[...],
                               preferred_element_type=jnp.float32)
    out_refs[3][...] = jnp.dot(hn, wnb_ref[...],
                               preferred_element_type=jnp.float32)
    if first:
      out_refs[4][...] = jnp.broadcast_to(cnt, cnt.shape[:1] + (PW,))


def _node_update(first, last, h, p16, mp0, mp1, pp0, pp1, cnt16, wu1a, wu1b,
                 bu1, gu1, blu1, wu2, bu2, gu2, blu2, mask3, wna, wnb, bn):
  grid = NP // NB
  full = lambda i: (0, 0)
  row = lambda i: (i, 0)
  rowp = pl.BlockSpec((NB, PW), row)
  rowps = pl.BlockSpec((NB, PS), row)
  rowd = pl.BlockSpec((NB, D), row)
  vec = pl.BlockSpec((1, D), full)
  if last:
    out_specs = [pl.BlockSpec((NB, 8), row)]
    out_shape = [jax.ShapeDtypeStruct((NP, 8), jnp.float32)]
  else:
    out_specs = [rowd, rowp, rowd, rowd]
    out_shape = [
        jax.ShapeDtypeStruct((NP, D), jnp.float32),
        jax.ShapeDtypeStruct((NP, PW), jnp.float32),
        jax.ShapeDtypeStruct((NP, D), jnp.float32),
        jax.ShapeDtypeStruct((NP, D), jnp.float32),
    ]
    if first:
      out_specs.append(rowp)
      out_shape.append(jax.ShapeDtypeStruct((NP, PW), jnp.float32))
  return pl.pallas_call(
      functools.partial(_upd_body, first, last),
      grid=(grid,),
      in_specs=[
          rowd, rowp, rowd, rowd, rowd, rowd, rowp,
          pl.BlockSpec((D, D), full), pl.BlockSpec((D, D), full),
          vec, vec, vec,
          pl.BlockSpec((D, D), full), vec, vec, vec,
          pl.BlockSpec((1, PW), full),
          pl.BlockSpec((D, D), full), pl.BlockSpec((D, D), full), vec,
      ],
      out_specs=out_specs,
      out_shape=out_shape,
  )(h, p16, mp0, mp1, pp0, pp1, cnt16, wu1a, wu1b, bu1, gu1, blu1, wu2, bu2,
    gu2, blu2, mask3, wna, wnb, bn)


def _row(v):
  return v.reshape(1, -1).astype(jnp.float32)


def kernel(z, time_features, pos, edge_index, params):
  f32 = jnp.float32
  x8 = jnp.zeros((NP, 8), f32)
  x8 = x8.at[:N, 0:1].set(z.astype(f32))
  x8 = x8.at[:N, 1].set(time_features.astype(f32))
  p16 = jnp.zeros((NP, PW), f32).at[:N, 0:3].set(pos.astype(f32))
  dst = edge_index[1].astype(jnp.int32)
  src = edge_index[0].astype(jnp.int32)

  convs = params["convs"]
  we8 = jnp.zeros((8, D), f32).at[0:2, :].set(params["emb_in"]["W"])
  be = _row(params["emb_in"]["b"])

  wa0 = convs[0]["msg1"]["W"][0:D, :]
  wb0 = convs[0]["msg1"]["W"][D:2 * D, :]
  h, A, B = _embed(x8, we8, be, wa0, wb0)

  mask3 = jnp.zeros((1, PW), f32).at[0, 0:3].set(1.0)
  cnt16 = jnp.zeros((NP, PW), f32)

  for li, lp in enumerate(convs):
    first = li == 0
    last = li == len(convs) - 1
    ppl = p16[:, 0:3].T.reshape(3 * NP)
    pre, pd = _gather(A, B, ppl, dst, src)
    one3 = jnp.zeros((1, PS), f32)
    if first:
      one3 = one3.at[0, 3].set(1.0)
    m, po = _edge_mlp(
        pre, pd,
        lp["msg2"]["W"], _row(lp["msg2"]["b"]),
        _row(lp["msg_ln1"]["g"]), _row(lp["msg_ln1"]["b"]),
        _row(lp["msg_ln2"]["g"]), _row(lp["msg_ln2"]["b"]),
        lp["pos1"]["W"], _row(lp["pos1"]["b"]),
        _row(lp["pos_ln"]["g"]), _row(lp["pos_ln"]["b"]),
        _row(lp["msg1"]["W"][2 * D, :]), _row(lp["msg1"]["b"]),
        _row(lp["pos2"]["W"][:, 0]),
        jnp.full((1, D), lp["pos2"]["b"][0] / D, f32),
        one3)
    mp, pp = _scatter(m, po, dst)
    if last:
      wna = params["pred1"]["W"]
      wnb = jnp.zeros((D, D), f32).at[:, 0:1].set(params["pred2"]["W"])
      bn = _row(params["pred1"]["b"])
    else:
      wna = convs[li + 1]["msg1"]["W"][0:D, :]
      wnb = convs[li + 1]["msg1"]["W"][D:2 * D, :]
      bn = jnp.zeros((1, D), f32)
    outs = _node_update(
        first, last, h, p16, mp[0], mp[1], pp[0], pp[1], cnt16,
        lp["upd1"]["W"][0:D, :], lp["upd1"]["W"][D:2 * D, :],
        _row(lp["upd1"]["b"]),
        _row(lp["upd_ln1"]["g"]), _row(lp["upd_ln1"]["b"]),
        lp["upd2"]["W"], _row(lp["upd2"]["b"]),
        _row(lp["upd_ln2"]["g"]), _row(lp["upd_ln2"]["b"]),
        mask3, wna, wnb, bn)
    if last:
      o8 = outs[0]
    elif first:
      h, p16, A, B, cnt16 = outs
    else:
      h, p16, A, B = outs

  return o8[:N, 0:1] + params["pred2"]["b"]
